# baseline (speedup 1.0000x reference)
import functools

import jax
import jax.numpy as jnp
from jax import lax
from jax.experimental import pallas as pl
from jax.experimental.pallas import tpu as pltpu
from jax.experimental.pallas import tpu_sc as plsc

_MAXP = 512
_EMB = 1024
_SEQ = 4096
_TOP = 2 * _MAXP
_NROW = _TOP + 1
_NC = 2
_NS = 16
_NW = _NC * _NS
_PREF = 1024
_RPW = _PREF // _NW
_CH = 8
_ICH = 16
_BROWS = 8
_LANES = 16
_TBLK = 512
_TGRID = (_SEQ - _PREF) // _TBLK


def _sc_prefix(off_hbm, table_hbm, out_hbm, off_v, idx_v, buf0, buf1, bcast,
               sem0, sem1, semw):
    wid = lax.axis_index("s") * _NC + lax.axis_index("c")
    base = wid * _RPW

    pltpu.sync_copy(off_hbm, off_v)
    s0 = base + off_v[...][0]

    aligned = jnp.bitwise_and(s0, 7) == 0
    whole_in = (s0 >= 0) & (s0 + _RPW - 1 <= _TOP)
    whole_cl = (s0 + _RPW - 1 <= 0) | (s0 >= _TOP)
    fast = whole_in & aligned

    @pl.when(fast)
    def _linear():
        s0a = pl.multiple_of(s0, 8)
        bufs = (buf0, buf1)
        sems = (sem0, sem1)
        nch = _RPW // _CH
        handles = [None] * nch

        def start(c):
            handles[c] = pltpu.async_copy(
                table_hbm.at[pl.ds(s0a + c * _CH, _CH)],
                bufs[c % 2].at[pl.ds(0, _CH)], sems[c % 2])

        start(0)
        start(1)
        for c in range(nch):
            handles[c].wait()
            pltpu.sync_copy(bufs[c % 2].at[pl.ds(0, _CH)],
                            out_hbm.at[pl.ds(base + c * _CH, _CH)])
            if c + 2 < nch:
                start(c + 2)

    @pl.when(whole_cl)
    def _broadcast():
        any_low = s0 + _RPW - 1 <= 0
        r_src = pl.multiple_of(jnp.where(any_low, 0, _TOP), 8)
        pltpu.sync_copy(table_hbm.at[pl.ds(r_src, 1)], bcast.at[pl.ds(0, 1)])
        for v in range(_EMB // _LANES):
            row0 = bcast[0, pl.ds(v * _LANES, _LANES)]
            for r in range(1, _BROWS):
                bcast[r, pl.ds(v * _LANES, _LANES)] = row0
        handles = [
            pltpu.async_copy(
                bcast, out_hbm.at[pl.ds(base + k * _BROWS, _BROWS)], semw)
            for k in range(_RPW // _BROWS)
        ]
        for h in handles:
            h.wait()

    @pl.when(jnp.logical_not(fast | whole_cl))
    def _general():
        iota = lax.iota(jnp.int32, _LANES)
        for j in range(_RPW // _LANES):
            vec = iota + j * _LANES + s0
            idx_v[pl.ds(j * _LANES, _LANES)] = (
                jnp.minimum(jnp.maximum(vec, 0), _TOP))
        bufs = (buf0, buf1)
        sems = (sem0, sem1)
        nch = _RPW // _ICH
        handles = [None] * nch

        def start(c):
            handles[c] = pltpu.async_copy(
                table_hbm.at[idx_v.at[pl.ds(c * _ICH, _ICH)]],
                bufs[c % 2], sems[c % 2])

        start(0)
        if nch > 1:
            start(1)
        for c in range(nch):
            handles[c].wait()
            pltpu.sync_copy(bufs[c % 2],
                            out_hbm.at[pl.ds(base + c * _ICH, _ICH)])
            if c + 2 < nch:
                start(c + 2)


def _tc_fill(s_ref, table_ref, pref_ref, out_ref):
    del pref_ref
    blk = pl.program_id(0)
    first = _PREF + blk * _TBLK + s_ref[0]

    aligned = jnp.bitwise_and(first, 7) == 0
    all_in = (first >= 0) & (first + _TBLK - 1 <= _TOP)
    all_cl = (first + _TBLK - 1 <= 0) | (first >= _TOP)
    fast_in = all_in & aligned

    @pl.when(all_cl)
    def _():
        r = jnp.where(first >= _TOP, _TOP // 8, 0) * 8
        row = table_ref[pl.ds(pl.multiple_of(r, 8), 1), :]
        out_ref[...] = jnp.broadcast_to(row, (_TBLK, _EMB))

    @pl.when(fast_in)
    def _():
        out_ref[...] = table_ref[pl.ds(pl.multiple_of(first, 8), _TBLK), :]

    @pl.when(jnp.logical_not(all_cl | fast_in))
    def _():
        rows = jnp.minimum(jnp.maximum(
            first + lax.broadcasted_iota(jnp.int32, (_TBLK, _NROW), 0), 0),
            _TOP)
        cols = lax.broadcasted_iota(jnp.int32, (_TBLK, _NROW), 1)
        onehot = (rows == cols).astype(jnp.float32)
        out_ref[...] = jax.lax.dot_general(
            onehot, table_ref[...], (((1,), (0,)), ((), ())),
            preferred_element_type=jnp.float32)


def kernel(seq_len_q, embeddings_table):
    s = jnp.asarray(seq_len_q, jnp.int32) - _SEQ + _MAXP
    off_vec = jnp.full((_LANES,), s, dtype=jnp.int32)
    table = embeddings_table.astype(jnp.float32)

    mesh = plsc.VectorSubcoreMesh(core_axis_name="c", subcore_axis_name="s")
    sc_run = functools.partial(
        pl.kernel,
        mesh=mesh,
        out_type=jax.ShapeDtypeStruct((_SEQ, _EMB), jnp.float32),
        scratch_types=[
            pltpu.VMEM((_LANES,), jnp.int32),
            pltpu.VMEM((_RPW,), jnp.int32),
            pltpu.VMEM((_ICH, _EMB), jnp.float32),
            pltpu.VMEM((_ICH, _EMB), jnp.float32),
            pltpu.VMEM((_BROWS, _EMB), jnp.float32),
            pltpu.SemaphoreType.DMA,
            pltpu.SemaphoreType.DMA,
            pltpu.SemaphoreType.DMA,
        ],
    )(_sc_prefix)
    prefix_out = sc_run(off_vec, table)

    out = pl.pallas_call(
        _tc_fill,
        grid=(_TGRID,),
        in_specs=[
            pl.BlockSpec(memory_space=pltpu.SMEM),
            pl.BlockSpec((_NROW, _EMB), lambda i: (0, 0)),
            pl.BlockSpec(memory_space=pl.ANY),
        ],
        out_specs=pl.BlockSpec((_TBLK, _EMB),
                               lambda i: (i + _PREF // _TBLK, 0)),
        out_shape=jax.ShapeDtypeStruct((_SEQ, _EMB), jnp.float32),
        input_output_aliases={2: 0},
    )(jnp.reshape(s, (1,)), table, prefix_out)
    return out
